# baseline (speedup 1.0000x reference)
import functools

import jax
import jax.numpy as jnp
from jax import lax
from jax.experimental import pallas as pl
from jax.experimental.pallas import tpu as pltpu
from jax.experimental.pallas import tpu_sc as plsc

N_ITEMS = 16384
N_DIMS = 32
NB = 128

LANES = 16
LC = 32
NGROUPS = LC // LANES
CHUNK = N_ITEMS // LC
RADIX = 256
NBINS = RADIX * LC
NPASS = 4


def _sort_body(lab_hbm, pred_hbm, out_hbm, lab_f32, key_a, key_b, val_a,
               val_b, hist):
    wid = lax.axis_index("c") * 16 + lax.axis_index("s")
    pltpu.sync_copy(lab_hbm.at[wid], lab_f32)
    pltpu.sync_copy(pred_hbm.at[wid], val_a)

    lane = lax.broadcasted_iota(jnp.int32, (LANES,), 0)
    ones = jnp.ones((LANES,), jnp.int32)

    def transform(i, _):
        k = lax.bitcast_convert_type(lab_f32[pl.ds(i * LANES, LANES)],
                                     jnp.int32)
        mask = (k >> 31) | jnp.int32(-2147483648)
        key_a[pl.ds(i * LANES, LANES)] = k ^ mask
        return 0

    lax.fori_loop(0, N_ITEMS // LANES, transform, 0)

    def one_pass(shift, src_key, src_val, dst_key, dst_val):
        def zero(j, _):
            hist[pl.ds(j * LANES, LANES)] = jnp.zeros((LANES,), jnp.int32)
            return 0

        lax.fori_loop(0, NBINS // LANES, zero, 0)

        def histo(i, _):
            for g in range(NGROUPS):
                idx = (g * LANES + lane) * CHUNK + i
                k = plsc.load_gather(src_key, [idx])
                digit = (lax.shift_right_logical(k, shift)
                         & jnp.int32(RADIX - 1))
                bin_ = digit * LC + (g * LANES) + lane
                plsc.addupdate_scatter(hist, [bin_], ones)
            return 0

        lax.fori_loop(0, CHUNK, histo, 0)

        def scan(j, carry):
            v = hist[pl.ds(j * LANES, LANES)]
            incl = plsc.cumsum(v)
            hist[pl.ds(j * LANES, LANES)] = incl - v + carry
            return carry + jnp.sum(v)

        lax.fori_loop(0, NBINS // LANES, scan, jnp.int32(0))

        def permute(i, _):
            for g in range(NGROUPS):
                idx = (g * LANES + lane) * CHUNK + i
                k = plsc.load_gather(src_key, [idx])
                v = plsc.load_gather(src_val, [idx])
                digit = (lax.shift_right_logical(k, shift)
                         & jnp.int32(RADIX - 1))
                bin_ = digit * LC + (g * LANES) + lane
                ofs = plsc.load_gather(hist, [bin_])
                plsc.store_scatter(dst_key, [ofs], k)
                plsc.store_scatter(dst_val, [ofs], v)
                plsc.addupdate_scatter(hist, [bin_], ones)
            return 0

        lax.fori_loop(0, CHUNK, permute, 0)

    one_pass(0, key_a, val_a, key_b, val_b)
    one_pass(8, key_b, val_b, key_a, val_a)
    one_pass(16, key_a, val_a, key_b, val_b)
    one_pass(24, key_b, val_b, key_a, val_a)

    pltpu.sync_copy(val_a, out_hbm.at[wid])


@functools.cache
def _sc_sort():
    return pl.kernel(
        _sort_body,
        out_type=jax.ShapeDtypeStruct((N_DIMS, N_ITEMS), jnp.float32),
        mesh=plsc.VectorSubcoreMesh(core_axis_name="c", subcore_axis_name="s"),
        compiler_params=pltpu.CompilerParams(needs_layout_passes=False),
        scratch_types=[
            pltpu.VMEM((N_ITEMS,), jnp.float32),
            pltpu.VMEM((N_ITEMS,), jnp.int32),
            pltpu.VMEM((N_ITEMS,), jnp.int32),
            pltpu.VMEM((N_ITEMS,), jnp.float32),
            pltpu.VMEM((N_ITEMS,), jnp.float32),
            pltpu.VMEM((NBINS,), jnp.int32),
        ],
    )


def _loss_body(sp_ref, out_ref):
    sp = sp_ref[...]
    m = jnp.max(sp, axis=1, keepdims=True)
    p = jnp.sum(sp, axis=1)
    e3 = jnp.exp(sp - m).reshape(N_DIMS, NB, NB)
    pos = lax.broadcasted_iota(jnp.int32, (NB, NB), 0)
    qix = lax.broadcasted_iota(jnp.int32, (NB, NB), 1)
    l_incl = (qix <= pos).astype(jnp.float32)
    l_strict = (qix < pos).astype(jnp.float32)
    within = lax.dot_general(
        e3, l_incl, (((2,), (1,)), ((), ())),
        preferred_element_type=jnp.float32)
    tot = jnp.sum(e3, axis=2)
    carry = lax.dot_general(
        tot, l_strict, (((1,), (1,)), ((), ())),
        preferred_element_type=jnp.float32)
    c = within + carry[:, :, None]
    term = jnp.sum(jnp.log(c))
    loss = (jnp.sum(N_ITEMS * m) - jnp.sum(p) + term) / N_DIMS
    out_ref[0, 0] = loss


@jax.jit
def kernel(predictions, labels):
    lab_t = labels.T
    pred_t = predictions.T
    sp = _sc_sort()(lab_t, pred_t)
    out = pl.pallas_call(
        _loss_body,
        out_shape=jax.ShapeDtypeStruct((1, 1), jnp.float32),
        in_specs=[pl.BlockSpec(memory_space=pltpu.VMEM)],
        out_specs=pl.BlockSpec(memory_space=pltpu.SMEM),
    )(sp)
    return out[0, 0]


# speedup vs baseline: 1.4494x; 1.4494x over previous
import functools

import jax
import jax.numpy as jnp
from jax import lax
from jax.experimental import pallas as pl
from jax.experimental.pallas import tpu as pltpu
from jax.experimental.pallas import tpu_sc as plsc

N_ITEMS = 16384
N_DIMS = 32
NB = 128

LANES = 16
LC = 32
NGROUPS = LC // LANES
CHUNK = N_ITEMS // LC
RADIX = 256
NBINS = RADIX * LC
NPASS = 4


HIST_UN = 2


def _sort_body(lab_hbm, pred_hbm, out_hbm, lab_f32, key_a, key_b, val_a,
               val_b, rank, hist, incl_buf, tot_excl):
    wid = lax.axis_index("c") * 16 + lax.axis_index("s")
    pltpu.sync_copy(lab_hbm.at[wid], lab_f32)
    pltpu.sync_copy(pred_hbm.at[wid], val_a)

    lane = lax.broadcasted_iota(jnp.int32, (LANES,), 0)
    ones = jnp.ones((LANES,), jnp.int32)

    @plsc.parallel_loop(0, N_ITEMS // LANES, unroll=4)
    def _(i):
        k = lax.bitcast_convert_type(lab_f32[pl.ds(i * LANES, LANES)],
                                     jnp.int32)
        mask = (k >> 31) | jnp.int32(-2147483648)
        key_a[pl.ds(i * LANES, LANES)] = k ^ mask

    def one_pass(shift, src_key, src_val, dst_key, dst_val):
        @plsc.parallel_loop(0, NBINS // LANES, unroll=8)
        def _(j):
            hist[pl.ds(j * LANES, LANES)] = jnp.zeros((LANES,), jnp.int32)

        def histo(i, _):
            for u in range(HIST_UN):
                i2 = i * HIST_UN + u
                for g in range(NGROUPS):
                    idx = (g * LANES + lane) * CHUNK + i2
                    k = plsc.load_gather(src_key, [idx])
                    digit = (lax.shift_right_logical(k, shift)
                             & jnp.int32(RADIX - 1))
                    bin_ = digit * LC + (g * LANES) + lane
                    c = plsc.load_gather(hist, [bin_])
                    rank[pl.ds(i2 * LC + g * LANES, LANES)] = c
                    plsc.addupdate_scatter(hist, [bin_], ones)
            return 0

        lax.fori_loop(0, CHUNK // HIST_UN, histo, 0)

        @plsc.parallel_loop(0, NBINS // LANES, unroll=4)
        def _(j):
            incl_buf[pl.ds(j * LANES, LANES)] = plsc.cumsum(
                hist[pl.ds(j * LANES, LANES)])

        def scan_tot(b, carry):
            tv = plsc.load_gather(incl_buf, [(b * LANES + lane) * LANES + 15])
            iv = plsc.cumsum(tv)
            tot_excl[pl.ds(b * LANES, LANES)] = iv - tv + carry
            return carry + jnp.squeeze(lax.slice(iv, (15,), (16,)))

        lax.fori_loop(0, NBINS // (LANES * LANES), scan_tot, jnp.int32(0))

        @plsc.parallel_loop(0, NBINS // LANES, unroll=4)
        def _(j):
            v = hist[pl.ds(j * LANES, LANES)]
            incl = incl_buf[pl.ds(j * LANES, LANES)]
            t = plsc.load_gather(tot_excl, [lane * 0 + j])
            hist[pl.ds(j * LANES, LANES)] = incl - v + t

        @plsc.parallel_loop(0, CHUNK, unroll=4)
        def _(i):
            for g in range(NGROUPS):
                idx = (g * LANES + lane) * CHUNK + i
                k = plsc.load_gather(src_key, [idx])
                v = plsc.load_gather(src_val, [idx])
                digit = (lax.shift_right_logical(k, shift)
                         & jnp.int32(RADIX - 1))
                bin_ = digit * LC + (g * LANES) + lane
                base = plsc.load_gather(hist, [bin_])
                r = rank[pl.ds(i * LC + g * LANES, LANES)]
                pos = base + r
                plsc.store_scatter(dst_key, [pos], k)
                plsc.store_scatter(dst_val, [pos], v)

    one_pass(0, key_a, val_a, key_b, val_b)
    one_pass(8, key_b, val_b, key_a, val_a)
    one_pass(16, key_a, val_a, key_b, val_b)
    one_pass(24, key_b, val_b, key_a, val_a)

    pltpu.sync_copy(val_a, out_hbm.at[wid])


@functools.cache
def _sc_sort():
    return pl.kernel(
        _sort_body,
        out_type=jax.ShapeDtypeStruct((N_DIMS, N_ITEMS), jnp.float32),
        mesh=plsc.VectorSubcoreMesh(core_axis_name="c", subcore_axis_name="s"),
        compiler_params=pltpu.CompilerParams(needs_layout_passes=False),
        scratch_types=[
            pltpu.VMEM((N_ITEMS,), jnp.float32),
            pltpu.VMEM((N_ITEMS,), jnp.int32),
            pltpu.VMEM((N_ITEMS,), jnp.int32),
            pltpu.VMEM((N_ITEMS,), jnp.float32),
            pltpu.VMEM((N_ITEMS,), jnp.float32),
            pltpu.VMEM((N_ITEMS,), jnp.int32),
            pltpu.VMEM((NBINS,), jnp.int32),
            pltpu.VMEM((NBINS,), jnp.int32),
            pltpu.VMEM((NBINS // LANES,), jnp.int32),
        ],
    )


def _loss_body(sp_ref, out_ref):
    sp = sp_ref[...]
    m = jnp.max(sp, axis=1, keepdims=True)
    p = jnp.sum(sp, axis=1)
    e3 = jnp.exp(sp - m).reshape(N_DIMS, NB, NB)
    pos = lax.broadcasted_iota(jnp.int32, (NB, NB), 0)
    qix = lax.broadcasted_iota(jnp.int32, (NB, NB), 1)
    l_incl = (qix <= pos).astype(jnp.float32)
    l_strict = (qix < pos).astype(jnp.float32)
    within = lax.dot_general(
        e3, l_incl, (((2,), (1,)), ((), ())),
        preferred_element_type=jnp.float32)
    tot = jnp.sum(e3, axis=2)
    carry = lax.dot_general(
        tot, l_strict, (((1,), (1,)), ((), ())),
        preferred_element_type=jnp.float32)
    c = within + carry[:, :, None]
    term = jnp.sum(jnp.log(c))
    loss = (jnp.sum(N_ITEMS * m) - jnp.sum(p) + term) / N_DIMS
    out_ref[0, 0] = loss


@jax.jit
def kernel(predictions, labels):
    lab_t = labels.T
    pred_t = predictions.T
    sp = _sc_sort()(lab_t, pred_t)
    out = pl.pallas_call(
        _loss_body,
        out_shape=jax.ShapeDtypeStruct((1, 1), jnp.float32),
        in_specs=[pl.BlockSpec(memory_space=pltpu.VMEM)],
        out_specs=pl.BlockSpec(memory_space=pltpu.SMEM),
    )(sp)
    return out[0, 0]
